# baseline (speedup 1.0000x reference)
import functools

import jax
import jax.numpy as jnp
from jax.experimental import pallas as pl
from jax.experimental.pallas import tpu as pltpu

_B, _N, _D = 16, 2048, 3
_TILE = 256
_BIG = 1e30


def _chamfer_tc_body(x_ref, yt_ref, inp_ref, outp_ref, acc_ref):
    x = x_ref[0]
    in_pid = inp_ref[0]
    out_pid = outp_ref[0]

    in_mask_c = in_pid != 0
    out_mask_r = out_pid != 0
    zero_mask_r = jnp.logical_not(out_mask_r)

    n_in = jnp.sum(in_mask_c.astype(jnp.float32))
    n_out = jnp.sum(out_mask_r.astype(jnp.float32))
    n_zero = jnp.float32(_N) - n_out

    y0 = yt_ref[0, 0:1, :]
    y1 = yt_ref[0, 1:2, :]
    y2 = yt_ref[0, 2:3, :]

    x0c = x[:, 0:1]
    x1c = x[:, 1:2]
    x2c = x[:, 2:3]
    x_norm = jnp.sqrt(x0c * x0c + x1c * x1c + x2c * x2c)
    x_norm_sum = jnp.sum(jnp.where(in_mask_c, x_norm, 0.0))

    y_norm = jnp.sqrt(y0 * y0 + y1 * y1 + y2 * y2)
    y_zero_sum = jnp.sum(jnp.where(zero_mask_r, y_norm, 0.0))

    pen_out = jnp.where(out_mask_r, 0.0, _BIG)

    colmin = jnp.full((1, _N), _BIG, dtype=jnp.float32)
    rowsum = jnp.float32(0.0)
    for t in range(_N // _TILE):
        sl = slice(t * _TILE, (t + 1) * _TILE)
        xs0 = x0c[sl, :]
        xs1 = x1c[sl, :]
        xs2 = x2c[sl, :]
        in_m_t = in_mask_c[sl, :]
        pen_in_t = jnp.where(in_m_t, 0.0, _BIG)

        d0 = xs0 - y0
        d1 = xs1 - y1
        d2 = xs2 - y2
        m = d0 * d0 + d1 * d1 + d2 * d2 + (pen_in_t + pen_out)

        row_min = jnp.min(m, axis=1, keepdims=True)
        rowsum += jnp.sum(jnp.where(in_m_t, jnp.sqrt(row_min), 0.0))
        colmin = jnp.minimum(colmin, jnp.min(m, axis=0, keepdims=True))

    sum_yx = jnp.sum(jnp.where(out_mask_r, jnp.sqrt(colmin), 0.0))

    n_in_part = jnp.maximum(1.0, n_in)
    n_out_part = jnp.maximum(1.0, n_out)
    n_zero_part = jnp.maximum(1.0, n_zero)

    chamfer = 0.5 * (rowsum / n_out_part + sum_yx / n_in_part)
    contrib = jnp.where(
        n_out == 0.0,
        x_norm_sum / n_in_part,
        jnp.where(n_in == 0.0, 0.0, chamfer),
    )
    ez = y_zero_sum / n_zero_part

    row_idx = jax.lax.broadcasted_iota(jnp.int32, (8, 128), 0)
    val = jnp.where(row_idx == 0, contrib, jnp.where(row_idx == 1, ez, 0.0))
    acc_ref[0] = val * (1.0 / _B)


def _chamfer_shard(target, reco, in_c, out_r):
    b_loc = target.shape[0]
    yt = jnp.transpose(reco, (0, 2, 1))

    acc = pl.pallas_call(
        _chamfer_tc_body,
        grid=(b_loc,),
        in_specs=[
            pl.BlockSpec((1, _N, _D), lambda b: (b, 0, 0)),
            pl.BlockSpec((1, _D, _N), lambda b: (b, 0, 0)),
            pl.BlockSpec((1, _N, 1), lambda b: (b, 0, 0)),
            pl.BlockSpec((1, 1, _N), lambda b: (b, 0, 0)),
        ],
        out_specs=pl.BlockSpec((1, 8, 128), lambda b: (b, 0, 0)),
        out_shape=jax.ShapeDtypeStruct((b_loc, 8, 128), jnp.float32),
        compiler_params=pltpu.CompilerParams(
            dimension_semantics=("parallel",)),
    )(target, yt, in_c, out_r)

    return jnp.sum(acc[:, 0, 0]), jnp.sum(acc[:, 1, 0])


@jax.jit
def kernel(target, reco, in_pid, out_pid):
    in_c = in_pid.astype(jnp.int32)[..., None]
    out_r = out_pid.astype(jnp.int32)[:, None, :]

    devs = jax.devices()
    ndev = 2 if (len(devs) >= 2 and _B % 2 == 0) else 1
    if ndev == 1:
        return _chamfer_shard(target, reco, in_c, out_r)

    mesh = jax.sharding.Mesh(devs[:ndev], ("d",))
    p_b = jax.sharding.PartitionSpec("d")
    p_r = jax.sharding.PartitionSpec()

    def _sharded(t, r, ic, orr):
        s0, s1 = _chamfer_shard(t, r, ic, orr)
        return (jax.lax.psum(s0, "d"), jax.lax.psum(s1, "d"))

    f = jax.shard_map(
        _sharded, mesh=mesh,
        in_specs=(p_b, p_b, p_b, p_b),
        out_specs=(p_r, p_r),
        check_vma=False,
    )
    return f(target, reco, in_c, out_r)


# speedup vs baseline: 4.3091x; 4.3091x over previous
import functools

import jax
import jax.numpy as jnp
from jax.experimental import pallas as pl
from jax.experimental.pallas import tpu as pltpu

_B, _N, _D = 16, 2048, 3
_TILE = 256
_BIG = 1e30


def _chamfer_tc_body(x_ref, yt_ref, inp_ref, outp_ref, acc_ref):
    x = x_ref[0]
    in_pid = inp_ref[0]
    out_pid = outp_ref[0]

    in_mask_c = in_pid != 0
    out_mask_r = out_pid != 0
    zero_mask_r = jnp.logical_not(out_mask_r)

    n_in = jnp.sum(in_mask_c.astype(jnp.float32))
    n_out = jnp.sum(out_mask_r.astype(jnp.float32))
    n_zero = jnp.float32(_N) - n_out

    y0 = yt_ref[0, 0:1, :]
    y1 = yt_ref[0, 1:2, :]
    y2 = yt_ref[0, 2:3, :]

    x0c = x[:, 0:1]
    x1c = x[:, 1:2]
    x2c = x[:, 2:3]
    xn2 = x0c * x0c + x1c * x1c + x2c * x2c
    x_norm = jnp.sqrt(xn2)
    x_norm_sum = jnp.sum(jnp.where(in_mask_c, x_norm, 0.0))

    yn2 = y0 * y0 + y1 * y1 + y2 * y2
    y_norm = jnp.sqrt(yn2)
    y_zero_sum = jnp.sum(jnp.where(zero_mask_r, y_norm, 0.0))

    c_row = yn2 + jnp.where(out_mask_r, 0.0, _BIG)
    a_col = xn2 + jnp.where(in_mask_c, 0.0, _BIG)

    m2x0 = -2.0 * x0c
    m2x1 = -2.0 * x1c
    m2x2 = -2.0 * x2c

    colmin = jnp.full((1, _N), _BIG, dtype=jnp.float32)
    rowsum = jnp.float32(0.0)
    for t in range(_N // _TILE):
        sl = slice(t * _TILE, (t + 1) * _TILE)
        base = a_col[sl, :] + c_row
        m = base + (m2x0[sl, :] * y0
                    + m2x1[sl, :] * y1
                    + m2x2[sl, :] * y2)

        in_m_t = in_mask_c[sl, :]
        row_min = jnp.min(m, axis=1, keepdims=True)
        row_min = jnp.maximum(row_min, 0.0)
        rowsum += jnp.sum(jnp.where(in_m_t, jnp.sqrt(row_min), 0.0))
        colmin = jnp.minimum(colmin, jnp.min(m, axis=0, keepdims=True))

    colmin = jnp.maximum(colmin, 0.0)
    sum_yx = jnp.sum(jnp.where(out_mask_r, jnp.sqrt(colmin), 0.0))

    n_in_part = jnp.maximum(1.0, n_in)
    n_out_part = jnp.maximum(1.0, n_out)
    n_zero_part = jnp.maximum(1.0, n_zero)

    chamfer = 0.5 * (rowsum / n_out_part + sum_yx / n_in_part)
    contrib = jnp.where(
        n_out == 0.0,
        x_norm_sum / n_in_part,
        jnp.where(n_in == 0.0, 0.0, chamfer),
    )
    ez = y_zero_sum / n_zero_part

    row_idx = jax.lax.broadcasted_iota(jnp.int32, (8, 128), 0)
    val = jnp.where(row_idx == 0, contrib, jnp.where(row_idx == 1, ez, 0.0))
    acc_ref[0] = val * (1.0 / _B)


@jax.jit
def kernel(target, reco, in_pid, out_pid):
    in_c = in_pid.astype(jnp.int32)[..., None]
    out_r = out_pid.astype(jnp.int32)[:, None, :]
    yt = jnp.transpose(reco, (0, 2, 1))

    acc = pl.pallas_call(
        _chamfer_tc_body,
        grid=(_B,),
        in_specs=[
            pl.BlockSpec((1, _N, _D), lambda b: (b, 0, 0)),
            pl.BlockSpec((1, _D, _N), lambda b: (b, 0, 0)),
            pl.BlockSpec((1, _N, 1), lambda b: (b, 0, 0)),
            pl.BlockSpec((1, 1, _N), lambda b: (b, 0, 0)),
        ],
        out_specs=pl.BlockSpec((1, 8, 128), lambda b: (b, 0, 0)),
        out_shape=jax.ShapeDtypeStruct((_B, 8, 128), jnp.float32),
    )(target, yt, in_c, out_r)

    return jnp.sum(acc[:, 0, 0]), jnp.sum(acc[:, 1, 0])


# speedup vs baseline: 6.3452x; 1.4725x over previous
import functools

import jax
import jax.numpy as jnp
from jax import lax
from jax.experimental import pallas as pl
from jax.experimental.pallas import tpu as pltpu
from jax.experimental.pallas import tpu_sc as plsc

_B, _N, _D = 16, 2048, 3
_TILE = 256
_BIG = 1e30
_SC_L = 16


def _chamfer_tc_body(x_ref, yt_ref, inp_ref, outp_ref, acc_ref):
    x = x_ref[0]
    in_pid = inp_ref[0]
    out_pid = outp_ref[0]

    in_mask_c = in_pid != 0
    out_mask_r = out_pid != 0

    n_in = jnp.sum(in_mask_c.astype(jnp.float32))
    n_out = jnp.sum(out_mask_r.astype(jnp.float32))

    x0c = x[:, 0:1]
    x1c = x[:, 1:2]
    x2c = x[:, 2:3]
    xn2f = x0c * x0c + x1c * x1c + x2c * x2c
    x_norm = jnp.sqrt(xn2f)
    x_norm_sum = jnp.sum(jnp.where(in_mask_c, x_norm, 0.0))

    xb = x.astype(jnp.bfloat16)
    ybt = yt_ref[0].astype(jnp.bfloat16)
    xbf0 = xb[:, 0:1].astype(jnp.float32)
    xbf1 = xb[:, 1:2].astype(jnp.float32)
    xbf2 = xb[:, 2:3].astype(jnp.float32)
    xn2 = xbf0 * xbf0 + xbf1 * xbf1 + xbf2 * xbf2
    ybf0 = ybt[0:1, :].astype(jnp.float32)
    ybf1 = ybt[1:2, :].astype(jnp.float32)
    ybf2 = ybt[2:3, :].astype(jnp.float32)
    yn2 = ybf0 * ybf0 + ybf1 * ybf1 + ybf2 * ybf2

    c_row = yn2 + jnp.where(out_mask_r, 0.0, _BIG)
    a_col = xn2 + jnp.where(in_mask_c, 0.0, _BIG)

    a_hi = a_col.astype(jnp.bfloat16)
    a_lo = (a_col - a_hi.astype(jnp.float32)).astype(jnp.bfloat16)
    c_hi = c_row.astype(jnp.bfloat16)
    c_lo = (c_row - c_hi.astype(jnp.float32)).astype(jnp.bfloat16)

    ones_c = jnp.ones((_N, 1), jnp.bfloat16)
    ones_r = jnp.ones((1, _N), jnp.bfloat16)
    xp = jnp.concatenate(
        [xb * jnp.bfloat16(-2.0), ones_c, ones_c, a_hi, a_lo],
        axis=1)
    yp = jnp.concatenate(
        [ybt, c_hi, c_lo, ones_r, ones_r], axis=0)

    colmin = jnp.full((1, _N), _BIG, dtype=jnp.float32)
    row_mins = []
    for t in range(_N // _TILE):
        sl = slice(t * _TILE, (t + 1) * _TILE)
        m = jax.lax.dot_general(
            xp[sl, :], yp,
            dimension_numbers=(((1,), (0,)), ((), ())),
            preferred_element_type=jnp.float32)

        row_mins.append(jnp.min(m, axis=1, keepdims=True))
        colmin = jnp.minimum(colmin, jnp.min(m, axis=0, keepdims=True))

    row_min_all = jnp.maximum(jnp.concatenate(row_mins, axis=0), 0.0)
    rowsum = jnp.sum(jnp.where(in_mask_c, jnp.sqrt(row_min_all), 0.0))

    colmin = jnp.maximum(colmin, 0.0)
    sum_yx = jnp.sum(jnp.where(out_mask_r, jnp.sqrt(colmin), 0.0))

    n_in_part = jnp.maximum(1.0, n_in)
    n_out_part = jnp.maximum(1.0, n_out)

    chamfer = 0.5 * (rowsum / n_out_part + sum_yx / n_in_part)
    contrib = jnp.where(
        n_out == 0.0,
        x_norm_sum / n_in_part,
        jnp.where(n_in == 0.0, 0.0, chamfer),
    )
    acc_ref[0] = jnp.full((8, 128), 1.0 / _B, jnp.float32) * contrib


def _sc_rsqrt(sq):
    i = lax.bitcast_convert_type(sq, jnp.int32)
    i = jnp.int32(0x5F3759DF) - (i >> 1)
    g = lax.bitcast_convert_type(i, jnp.float32)
    for _ in range(3):
        g = g * (1.5 - 0.5 * sq * g * g)
    return g


def _ez_sc_body(yt_hbm, pid_hbm, out_hbm, y_v, pid_v, res_v, sem):
    wid = lax.axis_index("s") * 2 + lax.axis_index("c")

    @pl.when(wid < _B)
    def _():
        pltpu.async_copy(yt_hbm.at[wid], y_v, sem).wait()
        pltpu.async_copy(pid_hbm.at[wid], pid_v, sem).wait()

        def body(i, carry):
            s, c = carry
            sl = pl.ds(i * _SC_L, _SC_L)
            y0 = y_v[0, sl]
            y1 = y_v[1, sl]
            y2 = y_v[2, sl]
            sq = jnp.maximum(y0 * y0 + y1 * y1 + y2 * y2, 1e-35)
            norm = sq * _sc_rsqrt(sq)
            mask = pid_v[sl] == 0
            s = s + jnp.where(mask, norm, 0.0)
            c = c + jnp.where(mask, 1.0, 0.0)
            return s, c

        zero = jnp.zeros((_SC_L,), jnp.float32)
        s, c = lax.fori_loop(0, _N // _SC_L, body, (zero, zero))
        num = zero + jnp.sum(s)
        den = zero + jnp.maximum(1.0, jnp.sum(c))
        res_v[...] = num / den
        pltpu.sync_copy(res_v, out_hbm.at[wid])


_ez_sc_kernel = functools.partial(
    pl.kernel,
    mesh=plsc.VectorSubcoreMesh(core_axis_name="c", subcore_axis_name="s"),
    compiler_params=pltpu.CompilerParams(needs_layout_passes=False),
    out_type=jax.ShapeDtypeStruct((_B, _SC_L), jnp.float32),
    scratch_types=[
        pltpu.VMEM((_D, _N), jnp.float32),
        pltpu.VMEM((_N,), jnp.int32),
        pltpu.VMEM((_SC_L,), jnp.float32),
        pltpu.SemaphoreType.DMA,
    ],
)(_ez_sc_body)


@jax.jit
def kernel(target, reco, in_pid, out_pid):
    in_c = in_pid.astype(jnp.int32)[..., None]
    out_r = out_pid.astype(jnp.int32)[:, None, :]
    yt = jnp.transpose(reco, (0, 2, 1))

    acc = pl.pallas_call(
        _chamfer_tc_body,
        grid=(_B,),
        in_specs=[
            pl.BlockSpec((1, _N, _D), lambda b: (b, 0, 0)),
            pl.BlockSpec((1, _D, _N), lambda b: (b, 0, 0)),
            pl.BlockSpec((1, _N, 1), lambda b: (b, 0, 0)),
            pl.BlockSpec((1, 1, _N), lambda b: (b, 0, 0)),
        ],
        out_specs=pl.BlockSpec((1, 8, 128), lambda b: (b, 0, 0)),
        out_shape=jax.ShapeDtypeStruct((_B, 8, 128), jnp.float32),
    )(target, yt, in_c, out_r)

    ez_rows = _ez_sc_kernel(yt, out_pid.astype(jnp.int32))

    return jnp.sum(acc[:, 0, 0]), jnp.mean(ez_rows[:, 0])
